# baseline (speedup 1.0000x reference)
import functools

import jax
import jax.numpy as jnp
from jax import lax
from jax.experimental import pallas as pl
from jax.experimental.pallas import tpu as pltpu
from jax.experimental.pallas import tpu_sc as plsc

NUM_WORDS = 50000
NUM_CHAR_FEATURES = 100000
FEATS_PER_WORD = 12
NUM_T = 75
BATCH = 1024
SEQ = 52

NC, NS, L = 2, 16, 16
NW = NC * NS

D = 80
_OFFS = tuple(range(0, D, L))

CW = 8
CR = CW * FEATS_PER_WORD
CHUNKS_A = 196
WPT = CW * CHUNKS_A
LAST_CHUNKS = (NUM_WORDS - 31 * WPT) // CW
NBUF = 4
PHASES = 2
PCHUNKS = CHUNKS_A // PHASES
PROWS = PCHUNKS * CW
OUTER_A = -(-PCHUNKS // NBUF)
ZROWS = 112
LAST_P1 = NUM_WORDS - 31 * WPT - PROWS

NTOK = BATCH * (SEQ - 2)
TPT = NTOK // NW
CT = 80
CHUNKS_C = TPT // CT

_MESH = plsc.VectorSubcoreMesh(core_axis_name="c", subcore_axis_name="s")
_SC_PARAMS = pltpu.CompilerParams(use_tc_tiling_on_sc=False)


@functools.partial(
    pl.kernel,
    out_type=jax.ShapeDtypeStruct((NUM_WORDS, D), jnp.float32),
    mesh=_MESH,
    scratch_types=[
        pltpu.VMEM((WPT * FEATS_PER_WORD,), jnp.int32),
        pltpu.VMEM((NBUF, CR, D), jnp.float32),
        pltpu.VMEM((ZROWS, D), jnp.float32),
        pltpu.VMEM((CR,), jnp.int32),
        pltpu.VMEM_SHARED((NS * PROWS, D), jnp.float32),
        pltpu.SemaphoreType.DMA((NBUF,)),
    ],
    compiler_params=_SC_PARAMS,
)
def _sc_bag(feats_hbm, weight_hbm, logits_hbm, idx_v, rows_v, zero_v, seg_v,
            acc_sh, gsem):
    cid = lax.axis_index("c")
    sid = lax.axis_index("s")
    wid = sid * NC + cid
    wbase0 = wid * WPT
    fbase0 = wbase0 * FEATS_PER_WORD
    nfull = WPT * FEATS_PER_WORD
    nlast = LAST_CHUNKS * CR
    last_chunk = jnp.where(wid == NW - 1, LAST_CHUNKS - 1, CHUNKS_A - 1)
    slab = sid * PROWS
    iota = lax.iota(jnp.int32, L)

    @pl.when(wid < NW - 1)
    def _():
        pltpu.sync_copy(feats_hbm.at[pl.ds(fbase0, nfull)], idx_v.at[pl.ds(0, nfull)])

    @pl.when(wid == NW - 1)
    def _():
        pltpu.sync_copy(feats_hbm.at[pl.ds(fbase0, nlast)], idx_v.at[pl.ds(0, nlast)])

    for r in range(ZROWS):
        for off in _OFFS:
            zero_v[r, pl.ds(off, L)] = jnp.zeros((L,), jnp.float32)

    def gather_start(ci, b):
        ce = jnp.minimum(ci, last_chunk)
        pltpu.async_copy(
            weight_hbm.at[idx_v.at[pl.ds(ce * CR, CR)]], rows_v.at[b], gsem.at[b]
        )

    for p in range(PHASES):
        for t in range(PROWS // ZROWS):
            pltpu.sync_copy(zero_v, acc_sh.at[pl.ds(slab + t * ZROWS, ZROWS)])
        for k in range(CR // L):
            seg_v[pl.ds(L * k, L)] = slab + lax.div(iota + L * k, FEATS_PER_WORD)
        for b in range(NBUF):
            gather_start(jnp.int32(p * PCHUNKS + b), b)

        def outer(i0, carry):
            for b in range(NBUF):
                lc = i0 * NBUF + b
                ci = p * PCHUNKS + lc
                pltpu.make_async_copy(
                    weight_hbm.at[idx_v.at[pl.ds(0, CR)]], rows_v.at[b], gsem.at[b]
                ).wait()

                @pl.when((lc < PCHUNKS) & (ci <= last_chunk))
                def _():
                    pltpu.sync_copy(rows_v.at[b], acc_sh.at[seg_v], add=True)

                for k in range(CR // L):
                    seg_v[pl.ds(L * k, L)] = seg_v[pl.ds(L * k, L)] + CW
                gather_start(ci + NBUF, b)
            return carry

        lax.fori_loop(0, OUTER_A, outer, 0)
        for b in range(NBUF):
            pltpu.make_async_copy(
                weight_hbm.at[idx_v.at[pl.ds(0, CR)]], rows_v.at[b], gsem.at[b]
            ).wait()
        out0 = wbase0 + p * PROWS
        if p == 0:
            pltpu.sync_copy(
                acc_sh.at[pl.ds(slab, PROWS)], logits_hbm.at[pl.ds(out0, PROWS)]
            )
        else:
            @pl.when(wid < NW - 1)
            def _():
                pltpu.sync_copy(
                    acc_sh.at[pl.ds(slab, PROWS)], logits_hbm.at[pl.ds(out0, PROWS)]
                )

            @pl.when(wid == NW - 1)
            def _():
                pltpu.sync_copy(
                    acc_sh.at[pl.ds(slab, LAST_P1)], logits_hbm.at[pl.ds(out0, LAST_P1)]
                )


_PAD_BLK = 4000


def _tc_pad_body(w_ref, out_ref):
    out_ref[:, :NUM_T] = w_ref[...]


_tc_pad = pl.pallas_call(
    _tc_pad_body,
    grid=(NUM_CHAR_FEATURES // _PAD_BLK,),
    in_specs=[pl.BlockSpec((_PAD_BLK, NUM_T), lambda i: (i, 0))],
    out_specs=pl.BlockSpec((_PAD_BLK, D), lambda i: (i, 0)),
    out_shape=jax.ShapeDtypeStruct((NUM_CHAR_FEATURES, D), jnp.float32),
)


def _tc_lse_body(logits_ref, out_ref):
    x = logits_ref[...]
    m = jnp.max(x, axis=0, keepdims=True)
    se = jnp.sum(jnp.exp(x - m), axis=0, keepdims=True)
    out_ref[...] = jnp.broadcast_to(m + jnp.log(se), (8, D))


_tc_lse = pl.pallas_call(
    _tc_lse_body,
    out_shape=jax.ShapeDtypeStruct((8, D), jnp.float32),
)


_POFFS = (0, 16, 32, 48, NUM_T - L)
NBUF_C = 2
PACK = CT * NUM_T


@functools.partial(
    pl.kernel,
    out_type=jax.ShapeDtypeStruct((NTOK * NUM_T,), jnp.float32),
    mesh=_MESH,
    scratch_types=[
        pltpu.VMEM((TPT // (SEQ - 2), SEQ), jnp.int32),
        pltpu.VMEM((TPT,), jnp.int32),
        pltpu.VMEM((NBUF_C, CT, D), jnp.float32),
        pltpu.VMEM((NBUF_C, PACK), jnp.float32),
        pltpu.VMEM((D,), jnp.float32),
        pltpu.SemaphoreType.DMA((NBUF_C,)),
        pltpu.SemaphoreType.DMA((NBUF_C,)),
    ],
    compiler_params=_SC_PARAMS,
)
def _sc_gather(lg_hbm, lse_hbm, words_hbm, out_hbm, raw_v, idx_v, rows_v,
               pack_v, lse_v, gsem, ssem):
    wid = lax.axis_index("s") * NC + lax.axis_index("c")
    tbase = wid * TPT
    rows_per_tile = TPT // (SEQ - 2)
    pltpu.sync_copy(words_hbm.at[pl.ds(wid * rows_per_tile, rows_per_tile)], raw_v)
    pltpu.sync_copy(lse_hbm.at[0], lse_v)
    for r in range(rows_per_tile):
        for off in (0, 16, 32, SEQ - 2 - L):
            idx_v[pl.ds((SEQ - 2) * r + off, L)] = raw_v[r, pl.ds(off + 1, L)]

    def gather_start(ci, b):
        ce = jnp.minimum(ci, CHUNKS_C - 1)
        pltpu.async_copy(
            lg_hbm.at[idx_v.at[pl.ds(ce * CT, CT)]], rows_v.at[b], gsem.at[b]
        )

    for b in range(NBUF_C):
        gather_start(jnp.int32(b), b)

    def outer(i0, carry):
        for b in range(NBUF_C):
            ci = i0 * NBUF_C + b
            pltpu.make_async_copy(
                lg_hbm.at[idx_v.at[pl.ds(0, CT)]], rows_v.at[b], gsem.at[b]
            ).wait()

            @pl.when(i0 > 0)
            def _():
                pltpu.make_async_copy(
                    pack_v.at[b], out_hbm.at[pl.ds(0, PACK)], ssem.at[b]
                ).wait()

            for r in range(CT):
                for off in _POFFS:
                    pack_v[b, pl.ds(NUM_T * r + off, L)] = (
                        rows_v[b, r, pl.ds(off, L)] - lse_v[pl.ds(off, L)]
                    )
            pltpu.async_copy(
                pack_v.at[b],
                out_hbm.at[pl.ds((tbase + ci * CT) * NUM_T, PACK)],
                ssem.at[b],
            )
            gather_start(ci + NBUF_C, b)
        return carry

    lax.fori_loop(0, CHUNKS_C // NBUF_C, outer, 0)
    for b in range(NBUF_C):
        pltpu.make_async_copy(
            lg_hbm.at[idx_v.at[pl.ds(0, CT)]], rows_v.at[b], gsem.at[b]
        ).wait()
        pltpu.make_async_copy(
            pack_v.at[b], out_hbm.at[pl.ds(0, PACK)], ssem.at[b]
        ).wait()


def kernel(words, all_words_char_features, offsets, weight):
    del offsets
    weight_pad = _tc_pad(weight)
    logits = _sc_bag(all_words_char_features, weight_pad)
    lse = _tc_lse(logits)
    out = _sc_gather(logits, lse, words)
    return out.reshape(BATCH, SEQ - 2, NUM_T)
